# Initial kernel scaffold; baseline (speedup 1.0000x reference)
import jax
import jax.numpy as jnp
from jax.experimental import pallas as pl


def kernel(x, edge_index, batch_number, W1, b1, W2, b2, W3, b3):
    raise NotImplementedError("write your pallas kernel here")


# speedup vs baseline: 54.9373x; 54.9373x over previous
import functools

import jax
import jax.numpy as jnp
from jax import lax
from jax.experimental import pallas as pl
from jax.experimental.pallas import tpu as pltpu
from jax.experimental.pallas import tpu_sc as plsc

NC = 2
NS = 16
FP = 8
CH = 1024
BLK = 2048
MAGIC = 0x5F3759DF


def _sc_aggregate(x_pad, src, dst, npad, epad):
    np16 = npad // NS
    xc = np16 // 8
    ec = epad // (NC * NS)
    deg_per_tile = epad // NS
    n_deg_chunks = deg_per_tile // CH
    n_edge_chunks = ec // CH

    mesh = plsc.VectorSubcoreMesh(core_axis_name="c", subcore_axis_name="s")

    @functools.partial(
        pl.kernel,
        out_type=(
            jax.ShapeDtypeStruct((NC * npad, FP), jnp.float32),
            jax.ShapeDtypeStruct((npad,), jnp.float32),
        ),
        mesh=mesh,
        compiler_params=pltpu.CompilerParams(
            needs_layout_passes=False, use_tc_tiling_on_sc=False),
        scratch_types=[
            pltpu.VMEM_SHARED((npad, FP), jnp.float32),
            pltpu.VMEM_SHARED((npad, FP), jnp.float32),
            pltpu.VMEM_SHARED((npad,), jnp.float32),
            pltpu.VMEM((np16,), jnp.float32),
            pltpu.VMEM((xc,), jnp.float32),
            pltpu.VMEM((CH,), jnp.int32),
            pltpu.VMEM((CH,), jnp.int32),
            pltpu.VMEM((CH, FP), jnp.float32),
        ],
    )
    def sc_kernel(x_hbm, src_hbm, dst_hbm, agg_hbm, dinv_hbm,
                  xs_sh, agg_sh, deg_sh,
                  ones_v, dinv_v, sidx_v, didx_v, rows_v):
        c = lax.axis_index("c")
        s = lax.axis_index("s")
        t0 = s * np16
        lane = lax.iota(jnp.int32, 16)

        def fill_ones(i, carry):
            ones_v[pl.ds(i * 16, 16)] = jnp.full((16,), 1.0, jnp.float32)
            return carry
        lax.fori_loop(0, np16 // 16, fill_ones, 0)
        pltpu.sync_copy(ones_v, deg_sh.at[pl.ds(t0, np16)])
        plsc.subcore_barrier()

        def deg_step(i, carry):
            off = s * deg_per_tile + i * CH
            pltpu.sync_copy(dst_hbm.at[pl.ds(off, CH)], didx_v)
            pltpu.sync_copy(ones_v.at[pl.ds(0, CH)], deg_sh.at[didx_v],
                            add=True)
            return carry
        lax.fori_loop(0, n_deg_chunks, deg_step, 0)
        plsc.subcore_barrier()

        rvec = lax.shift_right_arithmetic(lane, 3)
        cvec = lane & 7
        magic = jnp.full((16,), MAGIC, jnp.int32)

        def node_chunk(ci, carry):
            o = t0 + ci * xc
            pltpu.sync_copy(deg_sh.at[pl.ds(o, xc)], dinv_v)
            def rsq_step(i, carry2):
                y = dinv_v[pl.ds(i * 16, 16)]
                bi = magic - lax.shift_right_arithmetic(
                    plsc.bitcast(y, jnp.int32), 1)
                z = plsc.bitcast(bi, jnp.float32)
                z = z * (1.5 - 0.5 * y * z * z)
                z = z * (1.5 - 0.5 * y * z * z)
                z = z * (1.5 - 0.5 * y * z * z)
                dinv_v[pl.ds(i * 16, 16)] = z
                return carry2
            lax.fori_loop(0, xc // 16, rsq_step, 0)

            @pl.when(c == 0)
            def _():
                pltpu.sync_copy(dinv_v, dinv_hbm.at[pl.ds(o, xc)])

            pltpu.sync_copy(x_hbm.at[pl.ds(o, xc)],
                            rows_v.at[pl.ds(0, xc)])
            def xs_step(j, carry2):
                row = rvec + 2 * j
                d16 = plsc.load_gather(dinv_v, [row])
                v16 = plsc.load_gather(rows_v, [row, cvec])
                plsc.store_scatter(rows_v, [row, cvec], v16 * d16)
                return carry2
            lax.fori_loop(0, xc * FP // 16, xs_step, 0)
            pltpu.sync_copy(rows_v.at[pl.ds(0, xc)], xs_sh.at[pl.ds(o, xc)])
            pltpu.sync_copy(rows_v.at[pl.ds(0, xc)], agg_sh.at[pl.ds(o, xc)])
            return carry
        lax.fori_loop(0, np16 // xc, node_chunk, 0)
        plsc.subcore_barrier()

        wid = c * NS + s
        def edge_step(i, carry):
            off = wid * ec + i * CH
            pltpu.sync_copy(src_hbm.at[pl.ds(off, CH)], sidx_v)
            pltpu.sync_copy(dst_hbm.at[pl.ds(off, CH)], didx_v)
            pltpu.sync_copy(xs_sh.at[sidx_v], rows_v)
            pltpu.sync_copy(rows_v, agg_sh.at[didx_v], add=True)
            return carry
        lax.fori_loop(0, n_edge_chunks, edge_step, 0)
        plsc.subcore_barrier()

        pltpu.sync_copy(agg_sh.at[pl.ds(t0, np16)],
                        agg_hbm.at[pl.ds(c * npad + t0, np16)])

    return sc_kernel(x_pad, src, dst)


def _tc_dense(a0, a1, x_pad, dinv_col, bn3, W1p, b1r, W2, b2r, W3, b3r,
              npad, g, h, a):
    nb = npad // BLK

    def tc_body(a0_ref, a1_ref, x_ref, dv_ref, bn_ref,
                w1_ref, b1_ref, w2_ref, b2_ref, w3_ref, b3_ref,
                out_ref, sums_ref, cnt_ref):
        i = pl.program_id(0)

        @pl.when(i == 0)
        def _():
            sums_ref[...] = jnp.zeros_like(sums_ref)
            cnt_ref[...] = jnp.zeros_like(cnt_ref)

        d = dv_ref[...]
        node = (a0_ref[...] + a1_ref[...]) * d - (d * d) * x_ref[...]
        hblk = jnp.maximum(
            jnp.dot(node, w1_ref[...], preferred_element_type=jnp.float32)
            + b1_ref[...], 0.0)
        ids = bn_ref[0, 0, :]
        onehot = (lax.broadcasted_iota(jnp.int32, (g, BLK), 0)
                  == ids[None, :]).astype(jnp.float32)
        sums_ref[...] += jnp.dot(onehot, hblk,
                                 preferred_element_type=jnp.float32)
        cnt_ref[...] += jnp.sum(onehot, axis=1, keepdims=True)

        @pl.when(i == nb - 1)
        def _():
            mean = sums_ref[...] / jnp.maximum(cnt_ref[...], 1.0)
            h2 = jnp.maximum(
                jnp.dot(mean, w2_ref[...], preferred_element_type=jnp.float32)
                + b2_ref[...], 0.0)
            logits = jnp.dot(h2, w3_ref[...],
                             preferred_element_type=jnp.float32) + b3_ref[...]
            m = jnp.max(logits, axis=1, keepdims=True)
            lse = jnp.log(jnp.sum(jnp.exp(logits - m), axis=1,
                                  keepdims=True)) + m
            out_ref[...] = logits - lse

    full = lambda shape: pl.BlockSpec(shape, lambda i: (0,) * len(shape))
    return pl.pallas_call(
        tc_body,
        grid=(nb,),
        in_specs=[
            pl.BlockSpec((BLK, FP), lambda i: (i, 0)),
            pl.BlockSpec((BLK, FP), lambda i: (i, 0)),
            pl.BlockSpec((BLK, FP), lambda i: (i, 0)),
            pl.BlockSpec((BLK, 1), lambda i: (i, 0)),
            pl.BlockSpec((1, 1, BLK), lambda i: (i, 0, 0)),
            full((FP, h)), full((1, h)),
            full((h, h)), full((1, h)),
            full((h, a)), full((1, a)),
        ],
        out_specs=pl.BlockSpec((g, a), lambda i: (0, 0)),
        out_shape=jax.ShapeDtypeStruct((g, a), jnp.float32),
        scratch_shapes=[
            pltpu.VMEM((g, h), jnp.float32),
            pltpu.VMEM((g, 1), jnp.float32),
        ],
    )(a0, a1, x_pad, dinv_col, bn3, W1p, b1r, W2, b2r, W3, b3r)


def kernel(x, edge_index, batch_number, W1, b1, W2, b2, W3, b3):
    n, f_in = x.shape
    e = edge_index.shape[1]
    h = W1.shape[1]
    a = W3.shape[1]
    g = 256

    npad = -(-n // BLK) * BLK
    estep = NC * NS * CH
    epad = -(-e // estep) * estep

    x_pad = jnp.zeros((npad, FP), jnp.float32).at[:n, :f_in].set(x)
    pad_id = jnp.int32(npad - 1)
    src = jnp.concatenate(
        [edge_index[0], jnp.full((epad - e,), pad_id, jnp.int32)])
    dst = jnp.concatenate(
        [edge_index[1], jnp.full((epad - e,), pad_id, jnp.int32)])

    agg, dinv = _sc_aggregate(x_pad, src, dst, npad, epad)
    a0 = agg[:npad]
    a1 = agg[npad:]

    nb = npad // BLK
    bn3 = jnp.concatenate(
        [batch_number.astype(jnp.int32),
         jnp.full((npad - n,), jnp.int32(g), jnp.int32)]).reshape(nb, 1, BLK)
    W1p = jnp.zeros((FP, h), jnp.float32).at[:f_in].set(W1)

    return _tc_dense(a0, a1, x_pad, dinv.reshape(npad, 1), bn3,
                     W1p, b1.reshape(1, h), W2, b2.reshape(1, h),
                     W3, b3.reshape(1, a), npad, g, h, a)


# speedup vs baseline: 65.0856x; 1.1847x over previous
import functools

import jax
import jax.numpy as jnp
from jax import lax
from jax.experimental import pallas as pl
from jax.experimental.pallas import tpu as pltpu
from jax.experimental.pallas import tpu_sc as plsc

NC = 2
NS = 16
FP = 8
CH = 1024
BLK = 2048
MAGIC = 0x5F3759DF


def _sc_aggregate(x_pad, edge_index, npad, e, ch):
    np16 = npad // NS
    xc = np16 // 8
    ec = e // (NC * NS)
    deg_per_tile = e // NS
    n_deg_chunks = deg_per_tile // ch
    n_edge_chunks = ec // ch

    mesh = plsc.VectorSubcoreMesh(core_axis_name="c", subcore_axis_name="s")

    @functools.partial(
        pl.kernel,
        out_type=(
            jax.ShapeDtypeStruct((NC * npad, FP), jnp.float32),
            jax.ShapeDtypeStruct((npad,), jnp.float32),
        ),
        mesh=mesh,
        compiler_params=pltpu.CompilerParams(
            needs_layout_passes=False, use_tc_tiling_on_sc=False),
        scratch_types=[
            pltpu.VMEM_SHARED((npad, FP), jnp.float32),
            pltpu.VMEM_SHARED((npad, FP), jnp.float32),
            pltpu.VMEM_SHARED((npad,), jnp.float32),
            pltpu.VMEM((np16,), jnp.float32),
            pltpu.VMEM((xc,), jnp.float32),
            pltpu.VMEM((ch,), jnp.int32),
            pltpu.VMEM((ch,), jnp.int32),
            pltpu.VMEM((ch, FP), jnp.float32),
        ],
    )
    def sc_kernel(x_hbm, edge_hbm, agg_hbm, dinv_hbm,
                  xs_sh, agg_sh, deg_sh,
                  ones_v, dinv_v, sidx_v, didx_v, rows_v):
        c = lax.axis_index("c")
        s = lax.axis_index("s")
        t0 = s * np16
        lane = lax.iota(jnp.int32, 16)

        def fill_ones(i, carry):
            ones_v[pl.ds(i * 16, 16)] = jnp.full((16,), 1.0, jnp.float32)
            return carry
        lax.fori_loop(0, np16 // 16, fill_ones, 0)
        pltpu.sync_copy(ones_v, deg_sh.at[pl.ds(t0, np16)])
        plsc.subcore_barrier()

        def deg_step(i, carry):
            off = s * deg_per_tile + i * ch
            pltpu.sync_copy(edge_hbm.at[1, pl.ds(off, ch)], didx_v)
            pltpu.sync_copy(ones_v.at[pl.ds(0, ch)], deg_sh.at[didx_v],
                            add=True)
            return carry
        lax.fori_loop(0, n_deg_chunks, deg_step, 0)
        plsc.subcore_barrier()

        rvec = lax.shift_right_arithmetic(lane, 3)
        cvec = lane & 7
        magic = jnp.full((16,), MAGIC, jnp.int32)

        def node_chunk(ci, carry):
            o = t0 + ci * xc
            pltpu.sync_copy(deg_sh.at[pl.ds(o, xc)], dinv_v)
            def rsq_step(i, carry2):
                y = dinv_v[pl.ds(i * 16, 16)]
                bi = magic - lax.shift_right_arithmetic(
                    plsc.bitcast(y, jnp.int32), 1)
                z = plsc.bitcast(bi, jnp.float32)
                z = z * (1.5 - 0.5 * y * z * z)
                z = z * (1.5 - 0.5 * y * z * z)
                z = z * (1.5 - 0.5 * y * z * z)
                dinv_v[pl.ds(i * 16, 16)] = z
                return carry2
            lax.fori_loop(0, xc // 16, rsq_step, 0)

            @pl.when(c == 0)
            def _():
                pltpu.sync_copy(dinv_v, dinv_hbm.at[pl.ds(o, xc)])

            pltpu.sync_copy(x_hbm.at[pl.ds(o, xc)],
                            rows_v.at[pl.ds(0, xc)])
            def xs_step(j, carry2):
                row = rvec + 2 * j
                d16 = plsc.load_gather(dinv_v, [row])
                v16 = plsc.load_gather(rows_v, [row, cvec])
                plsc.store_scatter(rows_v, [row, cvec], v16 * d16)
                return carry2
            lax.fori_loop(0, xc * FP // 16, xs_step, 0)
            pltpu.sync_copy(rows_v.at[pl.ds(0, xc)], xs_sh.at[pl.ds(o, xc)])
            pltpu.sync_copy(rows_v.at[pl.ds(0, xc)], agg_sh.at[pl.ds(o, xc)])
            return carry
        lax.fori_loop(0, np16 // xc, node_chunk, 0)
        plsc.subcore_barrier()

        wid = c * NS + s
        def edge_step(i, carry):
            off = wid * ec + i * ch
            pltpu.sync_copy(edge_hbm.at[0, pl.ds(off, ch)], sidx_v)
            pltpu.sync_copy(edge_hbm.at[1, pl.ds(off, ch)], didx_v)
            pltpu.sync_copy(xs_sh.at[sidx_v], rows_v)
            pltpu.sync_copy(rows_v, agg_sh.at[didx_v], add=True)
            return carry
        lax.fori_loop(0, n_edge_chunks, edge_step, 0)
        plsc.subcore_barrier()

        pltpu.sync_copy(agg_sh.at[pl.ds(t0, np16)],
                        agg_hbm.at[pl.ds(c * npad + t0, np16)])

    return sc_kernel(x_pad, edge_index)


def _tc_dense(agg, x_pad, dinv_col, bn2, W1p, b1r, W2, b2r, W3, b3r,
              n, npad, g, h, a):
    nb = npad // BLK

    def tc_body(a0_ref, a1_ref, x_ref, dv_ref, bn_ref,
                w1_ref, b1_ref, w2_ref, b2_ref, w3_ref, b3_ref,
                out_ref, sums_ref, cnt_ref):
        i = pl.program_id(0)

        @pl.when(i == 0)
        def _():
            sums_ref[...] = jnp.zeros_like(sums_ref)
            cnt_ref[...] = jnp.zeros_like(cnt_ref)

        d = dv_ref[...]
        node = (a0_ref[...] + a1_ref[...]) * d - (d * d) * x_ref[...]
        hblk = jnp.maximum(
            jnp.dot(node, w1_ref[...], preferred_element_type=jnp.float32)
            + b1_ref[...], 0.0)
        ids = bn_ref[...]
        valid = (lax.broadcasted_iota(jnp.int32, (1, BLK), 1)
                 + i * BLK) < n
        onehot = ((lax.broadcasted_iota(jnp.int32, (g, BLK), 0) == ids)
                  & valid).astype(jnp.float32)
        sums_ref[...] += jnp.dot(onehot, hblk,
                                 preferred_element_type=jnp.float32)
        cnt_ref[...] += jnp.sum(onehot, axis=1, keepdims=True)

        @pl.when(i == nb - 1)
        def _():
            mean = sums_ref[...] / jnp.maximum(cnt_ref[...], 1.0)
            h2 = jnp.maximum(
                jnp.dot(mean, w2_ref[...], preferred_element_type=jnp.float32)
                + b2_ref[...], 0.0)
            logits = jnp.dot(h2, w3_ref[...],
                             preferred_element_type=jnp.float32) + b3_ref[...]
            m = jnp.max(logits, axis=1, keepdims=True)
            lse = jnp.log(jnp.sum(jnp.exp(logits - m), axis=1,
                                  keepdims=True)) + m
            out_ref[...] = logits - lse

    full = lambda shape: pl.BlockSpec(shape, lambda i: (0,) * len(shape))
    return pl.pallas_call(
        tc_body,
        grid=(nb,),
        in_specs=[
            pl.BlockSpec((BLK, FP), lambda i: (i, 0)),
            pl.BlockSpec((BLK, FP), lambda i: (i + nb, 0)),
            pl.BlockSpec((BLK, FP), lambda i: (i, 0)),
            pl.BlockSpec((BLK, 1), lambda i: (i, 0)),
            pl.BlockSpec((1, BLK), lambda i: (0, i)),
            full((FP, h)), full((1, h)),
            full((h, h)), full((1, h)),
            full((h, a)), full((1, a)),
        ],
        out_specs=pl.BlockSpec((g, a), lambda i: (0, 0)),
        out_shape=jax.ShapeDtypeStruct((g, a), jnp.float32),
        scratch_shapes=[
            pltpu.VMEM((g, h), jnp.float32),
            pltpu.VMEM((g, 1), jnp.float32),
        ],
    )(agg, agg, x_pad, dinv_col, bn2, W1p, b1r, W2, b2r, W3, b3r)


def kernel(x, edge_index, batch_number, W1, b1, W2, b2, W3, b3):
    n, f_in = x.shape
    e = edge_index.shape[1]
    h = W1.shape[1]
    a = W3.shape[1]
    g = 256

    npad = -(-n // BLK) * BLK
    e32 = e // (NC * NS)
    ch = next(c for c in range(1024, 0, -8) if e32 % c == 0)

    x_pad = jnp.zeros((npad, FP), jnp.float32).at[:n, :f_in].set(x)

    agg, dinv = _sc_aggregate(x_pad, edge_index, npad, e, ch)

    bn2 = batch_number.reshape(1, n)
    W1p = jnp.zeros((FP, h), jnp.float32).at[:f_in].set(W1)

    return _tc_dense(agg, x_pad, dinv.reshape(npad, 1), bn2,
                     W1p, b1.reshape(1, h), W2, b2.reshape(1, h),
                     W3, b3.reshape(1, a), n, npad, g, h, a)


# speedup vs baseline: 67.8037x; 1.0418x over previous
import functools

import jax
import jax.numpy as jnp
from jax import lax
from jax.experimental import pallas as pl
from jax.experimental.pallas import tpu as pltpu
from jax.experimental.pallas import tpu_sc as plsc

NC = 2
NS = 16
BLK = 2048
MAGIC = 0x5F3759DF


def _sc_aggregate(x, edge_index, n, f, npad, e, ch):
    FP = 8
    np16 = npad // NS
    xc = np16 // 16
    n_node_chunks = np16 // xc
    ec = e // (NC * NS)
    deg_per_tile = e // NS
    n_deg_chunks = deg_per_tile // ch
    n_edge_chunks = ec // ch
    flat = xc * FP

    mesh = plsc.VectorSubcoreMesh(core_axis_name="c", subcore_axis_name="s")

    @functools.partial(
        pl.kernel,
        out_type=jax.ShapeDtypeStruct((NC * npad, FP), jnp.float32),
        mesh=mesh,
        compiler_params=pltpu.CompilerParams(
            needs_layout_passes=False, use_tc_tiling_on_sc=False),
        scratch_types=[
            pltpu.VMEM_SHARED((npad, FP), jnp.float32),
            pltpu.VMEM_SHARED((npad, FP), jnp.float32),
            pltpu.VMEM_SHARED((npad,), jnp.float32),
            pltpu.VMEM((ch,), jnp.float32),
            pltpu.VMEM((np16,), jnp.float32),
            pltpu.VMEM((xc, f), jnp.float32),
            pltpu.VMEM((xc, FP), jnp.float32),
            pltpu.VMEM((ch,), jnp.int32),
            pltpu.VMEM((ch,), jnp.int32),
            pltpu.VMEM((ch, FP), jnp.float32),
        ],
    )
    def sc_kernel(x_hbm, edge_hbm, agg_hbm,
                  xs_sh, agg_sh, deg_sh,
                  ones_v, dinv_v, x6_v, x8_v, sidx_v, didx_v, rows_v):
        c = lax.axis_index("c")
        s = lax.axis_index("s")
        t0 = s * np16
        lane = lax.iota(jnp.int32, 16)
        fvec = jnp.full((16,), jnp.int32(f), jnp.int32)

        def fill_ones(i, carry):
            ones_v[pl.ds(i * 16, 16)] = jnp.full((16,), 1.0, jnp.float32)
            return carry
        lax.fori_loop(0, ch // 16, fill_ones, 0)
        for q in range(np16 // xc):
            pltpu.sync_copy(ones_v.at[pl.ds(0, xc)],
                            deg_sh.at[pl.ds(t0 + q * xc, xc)])
        plsc.subcore_barrier()

        def deg_step(i, carry):
            off = s * deg_per_tile + i * ch
            pltpu.sync_copy(edge_hbm.at[1, pl.ds(off, ch)], didx_v)
            pltpu.sync_copy(ones_v.at[pl.ds(0, ch)], deg_sh.at[didx_v],
                            add=True)
            return carry
        lax.fori_loop(0, n_deg_chunks, deg_step, 0)
        plsc.subcore_barrier()

        pltpu.sync_copy(deg_sh.at[pl.ds(t0, np16)], dinv_v)
        magic = jnp.full((16,), MAGIC, jnp.int32)
        def rsq_step(i, carry):
            y = dinv_v[pl.ds(i * 16, 16)]
            bi = magic - lax.shift_right_arithmetic(
                plsc.bitcast(y, jnp.int32), 1)
            z = plsc.bitcast(bi, jnp.float32)
            z = z * (1.5 - 0.5 * y * z * z)
            z = z * (1.5 - 0.5 * y * z * z)
            z = z * (1.5 - 0.5 * y * z * z)
            dinv_v[pl.ds(i * 16, 16)] = z
            return carry
        lax.fori_loop(0, np16 // 16, rsq_step, 0)

        zero16 = jnp.zeros((16,), jnp.float32)
        rr8 = lax.shift_right_arithmetic(lane, 3)
        cc8 = lane & 7

        def node_chunk(ci, carry):
            go = t0 + ci * xc
            lb = ci * xc

            @pl.when(go + xc <= n)
            def _():
                pltpu.sync_copy(x_hbm.at[pl.ds(go, xc)], x6_v)

            @pl.when(go + xc > n)
            def _():
                def zstep(j, carry2):
                    fi = j * 16 + lane
                    plsc.store_scatter(
                        x6_v, [lax.div(fi, fvec), lax.rem(fi, fvec)], zero16)
                    return carry2
                lax.fori_loop(0, (xc * f) // 16, zstep, 0)
                tail = n - (n // xc) * xc
                if tail:
                    pltpu.sync_copy(x_hbm.at[pl.ds((n // xc) * xc, tail)],
                                    x6_v.at[pl.ds(0, tail)])

            def xs_step(j, carry2):
                row = rr8 + 2 * j
                d16 = plsc.load_gather(dinv_v, [lb + row])
                v16 = plsc.load_gather(
                    x6_v, [row, jnp.minimum(cc8, jnp.int32(f - 1))])
                val = jnp.where(cc8 < f, v16 * d16, 0.0)
                plsc.store_scatter(x8_v, [row, cc8], val)
                return carry2
            lax.fori_loop(0, flat // 16, xs_step, 0)
            pltpu.sync_copy(x8_v, xs_sh.at[pl.ds(go, xc)])

            @pl.when(c == 0)
            def _():
                pltpu.sync_copy(x8_v, agg_sh.at[pl.ds(go, xc)])
            return carry
        lax.fori_loop(0, n_node_chunks, node_chunk, 0)

        @pl.when(c == 1)
        def _():
            def z8step(j, carry2):
                plsc.store_scatter(x8_v, [rr8 + 2 * j, cc8], zero16)
                return carry2
            lax.fori_loop(0, flat // 16, z8step, 0)
            def zc_step(ci, carry2):
                pltpu.sync_copy(x8_v, agg_sh.at[pl.ds(t0 + ci * xc, xc)])
                return carry2
            lax.fori_loop(0, n_node_chunks, zc_step, 0)
        plsc.subcore_barrier()

        wid = c * NS + s
        def edge_step(i, carry):
            off = wid * ec + i * ch
            pltpu.sync_copy(edge_hbm.at[0, pl.ds(off, ch)], sidx_v)
            pltpu.sync_copy(edge_hbm.at[1, pl.ds(off, ch)], didx_v)
            pltpu.sync_copy(xs_sh.at[sidx_v], rows_v)
            pltpu.sync_copy(rows_v, agg_sh.at[didx_v], add=True)
            return carry
        lax.fori_loop(0, n_edge_chunks, edge_step, 0)
        plsc.subcore_barrier()

        def out_chunk(ci, carry):
            go = t0 + ci * xc
            lb = ci * xc
            pltpu.sync_copy(agg_sh.at[pl.ds(go, xc)], x8_v)
            def sc_step(j, carry2):
                row = rr8 + 2 * j
                d16 = plsc.load_gather(dinv_v, [lb + row])
                v16 = plsc.load_gather(x8_v, [row, cc8])
                plsc.store_scatter(x8_v, [row, cc8], v16 * d16)
                return carry2
            lax.fori_loop(0, flat // 16, sc_step, 0)
            pltpu.sync_copy(x8_v, agg_hbm.at[pl.ds(c * npad + go, xc)])
            return carry
        lax.fori_loop(0, n_node_chunks, out_chunk, 0)

    return sc_kernel(x, edge_index)


def _tc_dense(agg, bn2, W1p, b1r, W2, b2r, W3, b3r, n, npad, g, h, a):
    nb = npad // BLK
    fp = W1p.shape[0]

    def tc_body(a0_ref, a1_ref, bn_ref,
                w1_ref, b1_ref, w2_ref, b2_ref, w3_ref, b3_ref,
                out_ref, sums_ref, cnt_ref):
        i = pl.program_id(0)

        @pl.when(i == 0)
        def _():
            sums_ref[...] = jnp.zeros_like(sums_ref)
            cnt_ref[...] = jnp.zeros_like(cnt_ref)

        node = a0_ref[...] + a1_ref[...]
        hblk = jnp.maximum(
            jnp.dot(node, w1_ref[...], preferred_element_type=jnp.float32)
            + b1_ref[...], 0.0)
        ids = bn_ref[...]
        valid = (lax.broadcasted_iota(jnp.int32, (1, BLK), 1)
                 + i * BLK) < n
        onehot = ((lax.broadcasted_iota(jnp.int32, (g, BLK), 0) == ids)
                  & valid).astype(jnp.float32)
        sums_ref[...] += jnp.dot(onehot, hblk,
                                 preferred_element_type=jnp.float32)
        cnt_ref[...] += jnp.sum(onehot, axis=1, keepdims=True)

        @pl.when(i == nb - 1)
        def _():
            mean = sums_ref[...] / jnp.maximum(cnt_ref[...], 1.0)
            h2 = jnp.maximum(
                jnp.dot(mean, w2_ref[...], preferred_element_type=jnp.float32)
                + b2_ref[...], 0.0)
            logits = jnp.dot(h2, w3_ref[...],
                             preferred_element_type=jnp.float32) + b3_ref[...]
            m = jnp.max(logits, axis=1, keepdims=True)
            lse = jnp.log(jnp.sum(jnp.exp(logits - m), axis=1,
                                  keepdims=True)) + m
            out_ref[...] = logits - lse

    full = lambda shape: pl.BlockSpec(shape, lambda i: (0,) * len(shape))
    return pl.pallas_call(
        tc_body,
        grid=(nb,),
        in_specs=[
            pl.BlockSpec((BLK, fp), lambda i: (i, 0)),
            pl.BlockSpec((BLK, fp), lambda i: (i + nb, 0)),
            pl.BlockSpec((1, BLK), lambda i: (0, i)),
            full((fp, h)), full((1, h)),
            full((h, h)), full((1, h)),
            full((h, a)), full((1, a)),
        ],
        out_specs=pl.BlockSpec((g, a), lambda i: (0, 0)),
        out_shape=jax.ShapeDtypeStruct((g, a), jnp.float32),
        scratch_shapes=[
            pltpu.VMEM((g, h), jnp.float32),
            pltpu.VMEM((g, 1), jnp.float32),
        ],
    )(agg, agg, bn2, W1p, b1r, W2, b2r, W3, b3r)


def kernel(x, edge_index, batch_number, W1, b1, W2, b2, W3, b3):
    n, f = x.shape
    e = edge_index.shape[1]
    h = W1.shape[1]
    a = W3.shape[1]
    g = 256

    npad = -(-n // BLK) * BLK
    e32 = e // (NC * NS)
    ch = next(c for c in range(1024, 0, -8) if e32 % c == 0)

    agg = _sc_aggregate(x, edge_index, n, f, npad, e, ch)
    W1p = jnp.zeros((8, h), jnp.float32).at[:f].set(W1)
    return _tc_dense(agg, batch_number.reshape(1, n), W1p,
                     b1.reshape(1, h), W2, b2.reshape(1, h),
                     W3, b3.reshape(1, a), n, npad, g, h, a)
